# baseline (speedup 1.0000x reference)
import jax
import jax.numpy as jnp
from jax.experimental import pallas as pl
from jax.experimental.pallas import tpu as pltpu

INPUT_DIM = 2048
LATENT_DIM = 16384
N_TOKENS = 8192
K = 32

R1 = 2048
L1 = 512


def _enc_kernel(x_ref, w_ref, b_ref, pre_ref):
    pre_ref[...] = jax.lax.dot_general(
        x_ref[...], w_ref[...], (((1,), (1,)), ((), ())),
        preferred_element_type=jnp.float32) + b_ref[...]


def _encode(x16, W16, b_enc2d):
    return pl.pallas_call(
        _enc_kernel,
        grid=(N_TOKENS // R1, LATENT_DIM // L1),
        in_specs=[
            pl.BlockSpec((R1, INPUT_DIM), lambda i, j: (i, 0)),
            pl.BlockSpec((L1, INPUT_DIM), lambda i, j: (j, 0)),
            pl.BlockSpec((1, L1), lambda i, j: (0, j)),
        ],
        out_specs=pl.BlockSpec((R1, L1), lambda i, j: (i, j)),
        out_shape=jax.ShapeDtypeStruct((N_TOKENS, LATENT_DIM), jnp.float32),
        compiler_params=pltpu.CompilerParams(
            dimension_semantics=("parallel", "parallel")),
    )(x16, W16, b_enc2d)


R2 = 128
G = 128
GW = LATENT_DIM // G
S = 40


def _bitsearch(u, nsteps):
    axes = tuple(range(1, u.ndim))

    def body(i, t):
        tb = t.reshape((R2,) + (1,) * (u.ndim - 1))
        cand = tb | (jnp.int32(1) << (jnp.int32(30) - i))
        cnt = jnp.sum((u >= cand).astype(jnp.int32), axis=axes)
        return jnp.where(cnt >= K, cand.reshape(R2), t)

    return jax.lax.fori_loop(0, nsteps, body, jnp.zeros((R2,), jnp.int32))


def _thresh_kernel(pre_ref, tau_ref, u_s):
    u_s[...] = (jax.lax.bitcast_convert_type(
        pre_ref[...], jnp.int32) & jnp.int32(0x7FFFFFFF)).reshape(R2, GW, G)
    u3 = u_s[...]
    g = jnp.max(u3, axis=1)
    gthr = _bitsearch(g, 31)
    q = g >= gthr[:, None]
    qi = q.astype(jnp.int32)
    cnt_q = jnp.sum(qi, axis=1)
    r_i = jax.lax.broadcasted_iota(jnp.int32, (G, G), 0)
    c_i = jax.lax.broadcasted_iota(jnp.int32, (G, G), 1)
    tril = (r_i < c_i).astype(jnp.bfloat16)
    pos = jax.lax.dot_general(
        qi.astype(jnp.bfloat16), tril, (((1,), (0,)), ((), ())),
        preferred_element_type=jnp.float32).astype(jnp.int32)
    lanes = jax.lax.broadcasted_iota(jnp.int32, (R2, G), 1)
    qid_cols = [
        jnp.sum(jnp.where(q & (pos == j), lanes, 0), axis=1, keepdims=True)
        for j in range(S)
    ]
    qid = jnp.concatenate(qid_cols, axis=1)
    idx3 = jnp.broadcast_to(qid[:, None, :], (R2, GW, S))
    cand = jnp.take_along_axis(u3, idx3, axis=2)
    slot = jax.lax.broadcasted_iota(jnp.int32, (R2, GW, S), 2)
    cand = jnp.where(slot < cnt_q[:, None, None], cand, 0)
    tau = _bitsearch(cand, 31)

    tau_ref[...] = jnp.broadcast_to(tau[:, None], (R2, 128))

    @pl.when(jnp.max(cnt_q) > S)
    def _slow():
        tau_ref[...] = jnp.broadcast_to(
            _bitsearch(u3, 31)[:, None], (R2, 128))


def _thresholds(pre):
    return pl.pallas_call(
        _thresh_kernel,
        grid=(N_TOKENS // R2,),
        in_specs=[pl.BlockSpec((R2, LATENT_DIM), lambda i: (i, 0))],
        out_specs=pl.BlockSpec((R2, 128), lambda i: (i, 0)),
        out_shape=jax.ShapeDtypeStruct((N_TOKENS, 128), jnp.int32),
        scratch_shapes=[pltpu.VMEM((R2, GW, G), jnp.int32)],
        compiler_params=pltpu.CompilerParams(
            dimension_semantics=("parallel",)),
    )(pre)


R3 = 1024
L3 = 512
NJ3 = LATENT_DIM // L3


def _dec_kernel(pre_ref, tau_ref, wd_ref, bd_ref, lat_ref, rec_ref):
    j = pl.program_id(1)
    pre = pre_ref[...]
    u = jax.lax.bitcast_convert_type(pre, jnp.int32) & jnp.int32(0x7FFFFFFF)
    lat = jnp.where(u >= tau_ref[:, 0:1], pre, 0.0)
    lat_ref[...] = lat

    partial = jax.lax.dot_general(
        lat.astype(jnp.bfloat16), wd_ref[...],
        (((1,), (1,)), ((), ())),
        preferred_element_type=jnp.float32)

    @pl.when(j == 0)
    def _init():
        rec_ref[...] = partial + bd_ref[...]

    @pl.when(j > 0)
    def _acc():
        rec_ref[...] = rec_ref[...] + partial


def _decode(pre, tau, Wd16, b_dec2d):
    return pl.pallas_call(
        _dec_kernel,
        grid=(N_TOKENS // R3, NJ3),
        in_specs=[
            pl.BlockSpec((R3, L3), lambda i, j: (i, j)),
            pl.BlockSpec((R3, 128), lambda i, j: (i, 0)),
            pl.BlockSpec((INPUT_DIM, L3), lambda i, j: (0, j)),
            pl.BlockSpec((1, INPUT_DIM), lambda i, j: (0, 0)),
        ],
        out_specs=[
            pl.BlockSpec((R3, L3), lambda i, j: (i, j)),
            pl.BlockSpec((R3, INPUT_DIM), lambda i, j: (i, 0)),
        ],
        out_shape=[
            jax.ShapeDtypeStruct((N_TOKENS, LATENT_DIM), jnp.float32),
            jax.ShapeDtypeStruct((N_TOKENS, INPUT_DIM), jnp.float32),
        ],
        compiler_params=pltpu.CompilerParams(
            dimension_semantics=("parallel", "arbitrary")),
    )(pre, tau, Wd16, b_dec2d)


@jax.jit
def kernel(x, W_enc, b_enc, W_dec, b_dec):
    x16 = x.astype(jnp.bfloat16)
    We16 = W_enc.astype(jnp.bfloat16)
    Wd16 = W_dec.astype(jnp.bfloat16)
    pre = _encode(x16, We16, b_enc.reshape(1, LATENT_DIM))
    tau = _thresholds(pre)
    latents, reconstructed = _decode(pre, tau, Wd16,
                                     b_dec.reshape(1, INPUT_DIM))
    return (reconstructed, latents, pre)


# speedup vs baseline: 1.7548x; 1.7548x over previous
import jax
import jax.numpy as jnp
from jax.experimental import pallas as pl
from jax.experimental.pallas import tpu as pltpu

INPUT_DIM = 2048
LATENT_DIM = 16384
N_TOKENS = 8192
K = 32

R1 = 2048
L1 = 512


def _enc_kernel(x_ref, w_ref, b_ref, pre_ref):
    pre_ref[...] = jax.lax.dot_general(
        x_ref[...], w_ref[...], (((1,), (1,)), ((), ())),
        preferred_element_type=jnp.float32) + b_ref[...]


def _encode(x16, W16, b_enc2d):
    return pl.pallas_call(
        _enc_kernel,
        grid=(N_TOKENS // R1, LATENT_DIM // L1),
        in_specs=[
            pl.BlockSpec((R1, INPUT_DIM), lambda i, j: (i, 0)),
            pl.BlockSpec((L1, INPUT_DIM), lambda i, j: (j, 0)),
            pl.BlockSpec((1, L1), lambda i, j: (0, j)),
        ],
        out_specs=pl.BlockSpec((R1, L1), lambda i, j: (i, j)),
        out_shape=jax.ShapeDtypeStruct((N_TOKENS, LATENT_DIM), jnp.float32),
        compiler_params=pltpu.CompilerParams(
            dimension_semantics=("parallel", "parallel")),
    )(x16, W16, b_enc2d)


R2 = 128
G = 128
GW = LATENT_DIM // G
S = 40


def _bitsearch(u, nsteps):
    axes = tuple(range(1, u.ndim))

    def body(i, t):
        tb = t.reshape((R2,) + (1,) * (u.ndim - 1))
        cand = tb | (jnp.int32(1) << (jnp.int32(30) - i))
        cnt = jnp.sum((u >= cand).astype(jnp.int32), axis=axes)
        return jnp.where(cnt >= K, cand.reshape(R2), t)

    return jax.lax.fori_loop(0, nsteps, body, jnp.zeros((R2,), jnp.int32))


def _thresh_kernel(pre_ref, tau_ref, u_s):
    u_s[...] = (jax.lax.bitcast_convert_type(
        pre_ref[...], jnp.int32) & jnp.int32(0x7FFFFFFF)).reshape(R2, GW, G)
    u3 = u_s[...]
    g = jnp.max(u3, axis=1)
    gthr = _bitsearch(g, 31)
    q = g >= gthr[:, None]
    qi = q.astype(jnp.int32)
    cnt_q = jnp.sum(qi, axis=1)
    r_i = jax.lax.broadcasted_iota(jnp.int32, (G, G), 0)
    c_i = jax.lax.broadcasted_iota(jnp.int32, (G, G), 1)
    tril = (r_i < c_i).astype(jnp.bfloat16)
    pos = jax.lax.dot_general(
        qi.astype(jnp.bfloat16), tril, (((1,), (0,)), ((), ())),
        preferred_element_type=jnp.float32).astype(jnp.int32)
    lanes = jax.lax.broadcasted_iota(jnp.int32, (R2, G), 1)
    qid_cols = [
        jnp.sum(jnp.where(q & (pos == j), lanes, 0), axis=1, keepdims=True)
        for j in range(S)
    ]
    qid = jnp.concatenate(qid_cols, axis=1)
    idx3 = jnp.broadcast_to(qid[:, None, :], (R2, GW, S))
    cand = jnp.take_along_axis(u3, idx3, axis=2)
    candt = jnp.swapaxes(cand, 1, 2)
    slot = jax.lax.broadcasted_iota(jnp.int32, (R2, S, GW), 1)
    candt = jnp.where(slot < cnt_q[:, None, None], candt, 0)
    tau = _bitsearch(candt, 31)

    tau_ref[...] = jnp.broadcast_to(tau[:, None], (R2, 128))

    @pl.when(jnp.max(cnt_q) > S)
    def _slow():
        tau_ref[...] = jnp.broadcast_to(
            _bitsearch(u3, 31)[:, None], (R2, 128))


def _thresholds(pre):
    return pl.pallas_call(
        _thresh_kernel,
        grid=(N_TOKENS // R2,),
        in_specs=[pl.BlockSpec((R2, LATENT_DIM), lambda i: (i, 0))],
        out_specs=pl.BlockSpec((R2, 128), lambda i: (i, 0)),
        out_shape=jax.ShapeDtypeStruct((N_TOKENS, 128), jnp.int32),
        scratch_shapes=[pltpu.VMEM((R2, GW, G), jnp.int32)],
        compiler_params=pltpu.CompilerParams(
            dimension_semantics=("parallel",)),
    )(pre)


R3 = 1024
L3 = 512
NJ3 = LATENT_DIM // L3


def _dec_kernel(pre_ref, tau_ref, wd_ref, bd_ref, lat_ref, rec_ref):
    j = pl.program_id(1)
    pre = pre_ref[...]
    u = jax.lax.bitcast_convert_type(pre, jnp.int32) & jnp.int32(0x7FFFFFFF)
    lat = jnp.where(u >= tau_ref[:, 0:1], pre, 0.0)
    lat_ref[...] = lat

    partial = jax.lax.dot_general(
        lat.astype(jnp.bfloat16), wd_ref[...],
        (((1,), (1,)), ((), ())),
        preferred_element_type=jnp.float32)

    @pl.when(j == 0)
    def _init():
        rec_ref[...] = partial + bd_ref[...]

    @pl.when(j > 0)
    def _acc():
        rec_ref[...] = rec_ref[...] + partial


def _decode(pre, tau, Wd16, b_dec2d):
    return pl.pallas_call(
        _dec_kernel,
        grid=(N_TOKENS // R3, NJ3),
        in_specs=[
            pl.BlockSpec((R3, L3), lambda i, j: (i, j)),
            pl.BlockSpec((R3, 128), lambda i, j: (i, 0)),
            pl.BlockSpec((INPUT_DIM, L3), lambda i, j: (0, j)),
            pl.BlockSpec((1, INPUT_DIM), lambda i, j: (0, 0)),
        ],
        out_specs=[
            pl.BlockSpec((R3, L3), lambda i, j: (i, j)),
            pl.BlockSpec((R3, INPUT_DIM), lambda i, j: (i, 0)),
        ],
        out_shape=[
            jax.ShapeDtypeStruct((N_TOKENS, LATENT_DIM), jnp.float32),
            jax.ShapeDtypeStruct((N_TOKENS, INPUT_DIM), jnp.float32),
        ],
        compiler_params=pltpu.CompilerParams(
            dimension_semantics=("parallel", "arbitrary")),
    )(pre, tau, Wd16, b_dec2d)


@jax.jit
def kernel(x, W_enc, b_enc, W_dec, b_dec):
    x16 = x.astype(jnp.bfloat16)
    We16 = W_enc.astype(jnp.bfloat16)
    Wd16 = W_dec.astype(jnp.bfloat16)
    pre = _encode(x16, We16, b_enc.reshape(1, LATENT_DIM))
    tau = _thresholds(pre)
    latents, reconstructed = _decode(pre, tau, Wd16,
                                     b_dec.reshape(1, INPUT_DIM))
    return (reconstructed, latents, pre)


# speedup vs baseline: 2.1901x; 1.2480x over previous
import jax
import jax.numpy as jnp
from jax.experimental import pallas as pl
from jax.experimental.pallas import tpu as pltpu

INPUT_DIM = 2048
LATENT_DIM = 16384
N_TOKENS = 8192
K = 32

R1 = 2048
L1 = 512


def _enc_kernel(x_ref, w_ref, b_ref, pre_ref):
    pre_ref[...] = jax.lax.dot_general(
        x_ref[...], w_ref[...], (((1,), (1,)), ((), ())),
        preferred_element_type=jnp.float32) + b_ref[...]


def _encode(x16, W16, b_enc2d):
    return pl.pallas_call(
        _enc_kernel,
        grid=(N_TOKENS // R1, LATENT_DIM // L1),
        in_specs=[
            pl.BlockSpec((R1, INPUT_DIM), lambda i, j: (i, 0)),
            pl.BlockSpec((L1, INPUT_DIM), lambda i, j: (j, 0)),
            pl.BlockSpec((1, L1), lambda i, j: (0, j)),
        ],
        out_specs=pl.BlockSpec((R1, L1), lambda i, j: (i, j)),
        out_shape=jax.ShapeDtypeStruct((N_TOKENS, LATENT_DIM), jnp.float32),
        compiler_params=pltpu.CompilerParams(
            dimension_semantics=("parallel", "parallel")),
    )(x16, W16, b_enc2d)


R2 = 256


def _bitsearch(u, nsteps):
    axes = tuple(range(1, u.ndim))

    def body(i, t):
        tb = t.reshape((R2,) + (1,) * (u.ndim - 1))
        cand = tb | (jnp.int32(1) << (jnp.int32(30) - i))
        cnt = jnp.sum((u >= cand).astype(jnp.int32), axis=axes)
        return jnp.where(cnt >= K, cand.reshape(R2), t)

    return jax.lax.fori_loop(0, nsteps, body, jnp.zeros((R2,), jnp.int32))


def _thresh_kernel(pre_ref, tau_ref, u_s):
    u_s[...] = jax.lax.bitcast_convert_type(
        pre_ref[...], jnp.int32) & jnp.int32(0x7FFFFFFF)
    tau = _bitsearch(u_s[...], 31)
    tau_ref[...] = jnp.broadcast_to(tau[:, None], (R2, 128))


def _thresholds(pre):
    return pl.pallas_call(
        _thresh_kernel,
        grid=(N_TOKENS // R2,),
        in_specs=[pl.BlockSpec((R2, LATENT_DIM), lambda i: (i, 0))],
        out_specs=pl.BlockSpec((R2, 128), lambda i: (i, 0)),
        out_shape=jax.ShapeDtypeStruct((N_TOKENS, 128), jnp.int32),
        scratch_shapes=[pltpu.VMEM((R2, LATENT_DIM), jnp.int32)],
        compiler_params=pltpu.CompilerParams(
            dimension_semantics=("parallel",)),
    )(pre)


R3 = 1024
L3 = 1024
NJ3 = LATENT_DIM // L3


def _dec_kernel(pre_ref, tau_ref, wd_ref, bd_ref, lat_ref, rec_ref):
    j = pl.program_id(1)
    pre = pre_ref[...]
    u = jax.lax.bitcast_convert_type(pre, jnp.int32) & jnp.int32(0x7FFFFFFF)
    lat = jnp.where(u >= tau_ref[:, 0:1], pre, 0.0)
    lat_ref[...] = lat

    partial = jax.lax.dot_general(
        lat.astype(jnp.bfloat16), wd_ref[...],
        (((1,), (1,)), ((), ())),
        preferred_element_type=jnp.float32)

    @pl.when(j == 0)
    def _init():
        rec_ref[...] = partial + bd_ref[...]

    @pl.when(j > 0)
    def _acc():
        rec_ref[...] = rec_ref[...] + partial


def _decode(pre, tau, Wd16, b_dec2d):
    return pl.pallas_call(
        _dec_kernel,
        grid=(N_TOKENS // R3, NJ3),
        in_specs=[
            pl.BlockSpec((R3, L3), lambda i, j: (i, j)),
            pl.BlockSpec((R3, 128), lambda i, j: (i, 0)),
            pl.BlockSpec((INPUT_DIM, L3), lambda i, j: (0, j)),
            pl.BlockSpec((1, INPUT_DIM), lambda i, j: (0, 0)),
        ],
        out_specs=[
            pl.BlockSpec((R3, L3), lambda i, j: (i, j)),
            pl.BlockSpec((R3, INPUT_DIM), lambda i, j: (i, 0)),
        ],
        out_shape=[
            jax.ShapeDtypeStruct((N_TOKENS, LATENT_DIM), jnp.float32),
            jax.ShapeDtypeStruct((N_TOKENS, INPUT_DIM), jnp.float32),
        ],
        compiler_params=pltpu.CompilerParams(
            dimension_semantics=("parallel", "arbitrary")),
    )(pre, tau, Wd16, b_dec2d)


@jax.jit
def kernel(x, W_enc, b_enc, W_dec, b_dec):
    x16 = x.astype(jnp.bfloat16)
    We16 = W_enc.astype(jnp.bfloat16)
    Wd16 = W_dec.astype(jnp.bfloat16)
    pre = _encode(x16, We16, b_enc.reshape(1, LATENT_DIM))
    tau = _thresholds(pre)
    latents, reconstructed = _decode(pre, tau, Wd16,
                                     b_dec.reshape(1, INPUT_DIM))
    return (reconstructed, latents, pre)


# speedup vs baseline: 2.1902x; 1.0000x over previous
import jax
import jax.numpy as jnp
from jax.experimental import pallas as pl
from jax.experimental.pallas import tpu as pltpu

INPUT_DIM = 2048
LATENT_DIM = 16384
N_TOKENS = 8192
K = 32

R1 = 2048
L1 = 512


def _enc_kernel(x_ref, w_ref, b_ref, pre_ref):
    pre_ref[...] = jax.lax.dot_general(
        x_ref[...], w_ref[...], (((1,), (1,)), ((), ())),
        preferred_element_type=jnp.float32) + b_ref[...]


def _encode(x16, W16, b_enc2d):
    return pl.pallas_call(
        _enc_kernel,
        grid=(N_TOKENS // R1, LATENT_DIM // L1),
        in_specs=[
            pl.BlockSpec((R1, INPUT_DIM), lambda i, j: (i, 0)),
            pl.BlockSpec((L1, INPUT_DIM), lambda i, j: (j, 0)),
            pl.BlockSpec((1, L1), lambda i, j: (0, j)),
        ],
        out_specs=pl.BlockSpec((R1, L1), lambda i, j: (i, j)),
        out_shape=jax.ShapeDtypeStruct((N_TOKENS, LATENT_DIM), jnp.float32),
        compiler_params=pltpu.CompilerParams(
            dimension_semantics=("parallel", "parallel")),
    )(x16, W16, b_enc2d)


R2 = 128

_NET8 = ((0, 1), (2, 3), (4, 5), (6, 7), (0, 2), (1, 3), (4, 6), (5, 7),
         (1, 2), (5, 6), (0, 4), (1, 5), (2, 6), (3, 7), (2, 4), (3, 5),
         (1, 2), (3, 4), (5, 6))


def _bitsearch(u, nsteps):
    axes = tuple(range(1, u.ndim))

    def body(i, t):
        tb = t.reshape((R2,) + (1,) * (u.ndim - 1))
        cand = tb | (jnp.int32(1) << (jnp.int32(30) - i))
        cnt = jnp.sum((u >= cand).astype(jnp.int32), axis=axes)
        return jnp.where(cnt >= K, cand.reshape(R2), t)

    return jax.lax.fori_loop(0, nsteps, body, jnp.zeros((R2,), jnp.int32))


def _thresh_kernel(pre_ref, tau_ref, u_s):
    u_s[...] = jax.lax.bitcast_convert_type(
        pre_ref[...], jnp.int32) & jnp.int32(0x7FFFFFFF)
    u = u_s[...]
    u3 = u.reshape(R2, 128, 128)
    e = [u3[:, 16 * j:16 * (j + 1), :] for j in range(8)]
    for (i, j) in _NET8:
        hi = jnp.maximum(e[i], e[j])
        lo = jnp.minimum(e[i], e[j])
        e[i], e[j] = hi, lo
    while e[0].shape[1] > 1:
        h = e[0].shape[1] // 2
        A = [c[:, :h] for c in e]
        B = [c[:, h:] for c in e]
        C = [jnp.maximum(A[i], B[7 - i]) for i in range(8)]
        for d in (4, 2, 1):
            for s0 in range(0, 8, 2 * d):
                for o in range(d):
                    i, j = s0 + o, s0 + o + d
                    hi = jnp.maximum(C[i], C[j])
                    lo = jnp.minimum(C[i], C[j])
                    C[i], C[j] = hi, lo
        e = C
    cand = jnp.concatenate(e, axis=1)
    tau = _bitsearch(cand, 31)
    tau_ref[...] = jnp.broadcast_to(tau[:, None], (R2, 128))

    ok = jnp.all(e[7][:, 0, :] < tau[:, None])

    @pl.when(jnp.logical_not(ok))
    def _slow():
        tau_ref[...] = jnp.broadcast_to(
            _bitsearch(u, 31)[:, None], (R2, 128))


def _thresholds(pre):
    return pl.pallas_call(
        _thresh_kernel,
        grid=(N_TOKENS // R2,),
        in_specs=[pl.BlockSpec((R2, LATENT_DIM), lambda i: (i, 0))],
        out_specs=pl.BlockSpec((R2, 128), lambda i: (i, 0)),
        out_shape=jax.ShapeDtypeStruct((N_TOKENS, 128), jnp.int32),
        scratch_shapes=[pltpu.VMEM((R2, LATENT_DIM), jnp.int32)],
        compiler_params=pltpu.CompilerParams(
            dimension_semantics=("parallel",)),
    )(pre)


R3 = 1024
L3 = 1024
NJ3 = LATENT_DIM // L3


def _dec_kernel(pre_ref, tau_ref, wd_ref, bd_ref, lat_ref, rec_ref):
    j = pl.program_id(1)
    pre = pre_ref[...]
    u = jax.lax.bitcast_convert_type(pre, jnp.int32) & jnp.int32(0x7FFFFFFF)
    lat = jnp.where(u >= tau_ref[:, 0:1], pre, 0.0)
    lat_ref[...] = lat

    partial = jax.lax.dot_general(
        lat.astype(jnp.bfloat16), wd_ref[...],
        (((1,), (1,)), ((), ())),
        preferred_element_type=jnp.float32)

    @pl.when(j == 0)
    def _init():
        rec_ref[...] = partial + bd_ref[...]

    @pl.when(j > 0)
    def _acc():
        rec_ref[...] = rec_ref[...] + partial


def _decode(pre, tau, Wd16, b_dec2d):
    return pl.pallas_call(
        _dec_kernel,
        grid=(N_TOKENS // R3, NJ3),
        in_specs=[
            pl.BlockSpec((R3, L3), lambda i, j: (i, j)),
            pl.BlockSpec((R3, 128), lambda i, j: (i, 0)),
            pl.BlockSpec((INPUT_DIM, L3), lambda i, j: (0, j)),
            pl.BlockSpec((1, INPUT_DIM), lambda i, j: (0, 0)),
        ],
        out_specs=[
            pl.BlockSpec((R3, L3), lambda i, j: (i, j)),
            pl.BlockSpec((R3, INPUT_DIM), lambda i, j: (i, 0)),
        ],
        out_shape=[
            jax.ShapeDtypeStruct((N_TOKENS, LATENT_DIM), jnp.float32),
            jax.ShapeDtypeStruct((N_TOKENS, INPUT_DIM), jnp.float32),
        ],
        compiler_params=pltpu.CompilerParams(
            dimension_semantics=("parallel", "arbitrary")),
    )(pre, tau, Wd16, b_dec2d)


@jax.jit
def kernel(x, W_enc, b_enc, W_dec, b_dec):
    x16 = x.astype(jnp.bfloat16)
    We16 = W_enc.astype(jnp.bfloat16)
    Wd16 = W_dec.astype(jnp.bfloat16)
    pre = _encode(x16, We16, b_enc.reshape(1, LATENT_DIM))
    tau = _thresholds(pre)
    latents, reconstructed = _decode(pre, tau, Wd16,
                                     b_dec.reshape(1, INPUT_DIM))
    return (reconstructed, latents, pre)


# speedup vs baseline: 2.2073x; 1.0078x over previous
import jax
import jax.numpy as jnp
from jax.experimental import pallas as pl
from jax.experimental.pallas import tpu as pltpu

INPUT_DIM = 2048
LATENT_DIM = 16384
N_TOKENS = 8192
K = 32

R1 = 2048
L1 = 1024


def _enc_kernel(x_ref, w_ref, b_ref, pre_ref):
    pre_ref[...] = jax.lax.dot_general(
        x_ref[...], w_ref[...], (((1,), (1,)), ((), ())),
        preferred_element_type=jnp.float32) + b_ref[...]


def _encode(x16, W16, b_enc2d):
    return pl.pallas_call(
        _enc_kernel,
        grid=(N_TOKENS // R1, LATENT_DIM // L1),
        in_specs=[
            pl.BlockSpec((R1, INPUT_DIM), lambda i, j: (i, 0)),
            pl.BlockSpec((L1, INPUT_DIM), lambda i, j: (j, 0)),
            pl.BlockSpec((1, L1), lambda i, j: (0, j)),
        ],
        out_specs=pl.BlockSpec((R1, L1), lambda i, j: (i, j)),
        out_shape=jax.ShapeDtypeStruct((N_TOKENS, LATENT_DIM), jnp.float32),
        compiler_params=pltpu.CompilerParams(
            dimension_semantics=("parallel", "parallel")),
    )(x16, W16, b_enc2d)


R2 = 128

_NET8 = ((0, 1), (2, 3), (4, 5), (6, 7), (0, 2), (1, 3), (4, 6), (5, 7),
         (1, 2), (5, 6), (0, 4), (1, 5), (2, 6), (3, 7), (2, 4), (3, 5),
         (1, 2), (3, 4), (5, 6))


def _bitsearch(u, nsteps):
    axes = tuple(range(1, u.ndim))

    def body(i, t):
        tb = t.reshape((R2,) + (1,) * (u.ndim - 1))
        cand = tb | (jnp.int32(1) << (jnp.int32(30) - i))
        cnt = jnp.sum((u >= cand).astype(jnp.int32), axis=axes)
        return jnp.where(cnt >= K, cand.reshape(R2), t)

    return jax.lax.fori_loop(0, nsteps, body, jnp.zeros((R2,), jnp.int32))


def _thresh_kernel(pre_ref, tau_ref, u_s):
    u_s[...] = jax.lax.bitcast_convert_type(
        pre_ref[...], jnp.int32) & jnp.int32(0x7FFFFFFF)
    u = u_s[...]
    u3 = u.reshape(R2, 128, 128)
    e = [u3[:, 16 * j:16 * (j + 1), :] for j in range(8)]
    for (i, j) in _NET8:
        hi = jnp.maximum(e[i], e[j])
        lo = jnp.minimum(e[i], e[j])
        e[i], e[j] = hi, lo
    while e[0].shape[1] > 1:
        h = e[0].shape[1] // 2
        A = [c[:, :h] for c in e]
        B = [c[:, h:] for c in e]
        C = [jnp.maximum(A[i], B[7 - i]) for i in range(8)]
        for d in (4, 2, 1):
            for s0 in range(0, 8, 2 * d):
                for o in range(d):
                    i, j = s0 + o, s0 + o + d
                    hi = jnp.maximum(C[i], C[j])
                    lo = jnp.minimum(C[i], C[j])
                    C[i], C[j] = hi, lo
        e = C
    cand = jnp.concatenate(e, axis=1)
    tau = _bitsearch(cand, 31)
    tau_ref[...] = jnp.broadcast_to(tau[:, None], (R2, 128))

    ok = jnp.all(e[7][:, 0, :] < tau[:, None])

    @pl.when(jnp.logical_not(ok))
    def _slow():
        tau_ref[...] = jnp.broadcast_to(
            _bitsearch(u, 31)[:, None], (R2, 128))


def _thresholds(pre):
    return pl.pallas_call(
        _thresh_kernel,
        grid=(N_TOKENS // R2,),
        in_specs=[pl.BlockSpec((R2, LATENT_DIM), lambda i: (i, 0))],
        out_specs=pl.BlockSpec((R2, 128), lambda i: (i, 0)),
        out_shape=jax.ShapeDtypeStruct((N_TOKENS, 128), jnp.int32),
        scratch_shapes=[pltpu.VMEM((R2, LATENT_DIM), jnp.int32)],
        compiler_params=pltpu.CompilerParams(
            dimension_semantics=("parallel",)),
    )(pre)


R3 = 1024
L3 = 1024
NJ3 = LATENT_DIM // L3


def _dec_kernel(pre_ref, tau_ref, wd_ref, bd_ref, lat_ref, rec_ref):
    j = pl.program_id(1)
    pre = pre_ref[...]
    u = jax.lax.bitcast_convert_type(pre, jnp.int32) & jnp.int32(0x7FFFFFFF)
    lat = jnp.where(u >= tau_ref[:, 0:1], pre, 0.0)
    lat_ref[...] = lat

    partial = jax.lax.dot_general(
        lat.astype(jnp.bfloat16), wd_ref[...],
        (((1,), (1,)), ((), ())),
        preferred_element_type=jnp.float32)

    @pl.when(j == 0)
    def _init():
        rec_ref[...] = partial + bd_ref[...]

    @pl.when(j > 0)
    def _acc():
        rec_ref[...] = rec_ref[...] + partial


def _decode(pre, tau, Wd16, b_dec2d):
    return pl.pallas_call(
        _dec_kernel,
        grid=(N_TOKENS // R3, NJ3),
        in_specs=[
            pl.BlockSpec((R3, L3), lambda i, j: (i, j)),
            pl.BlockSpec((R3, 128), lambda i, j: (i, 0)),
            pl.BlockSpec((INPUT_DIM, L3), lambda i, j: (0, j)),
            pl.BlockSpec((1, INPUT_DIM), lambda i, j: (0, 0)),
        ],
        out_specs=[
            pl.BlockSpec((R3, L3), lambda i, j: (i, j)),
            pl.BlockSpec((R3, INPUT_DIM), lambda i, j: (i, 0)),
        ],
        out_shape=[
            jax.ShapeDtypeStruct((N_TOKENS, LATENT_DIM), jnp.float32),
            jax.ShapeDtypeStruct((N_TOKENS, INPUT_DIM), jnp.float32),
        ],
        compiler_params=pltpu.CompilerParams(
            dimension_semantics=("parallel", "arbitrary")),
    )(pre, tau, Wd16, b_dec2d)


@jax.jit
def kernel(x, W_enc, b_enc, W_dec, b_dec):
    x16 = x.astype(jnp.bfloat16)
    We16 = W_enc.astype(jnp.bfloat16)
    Wd16 = W_dec.astype(jnp.bfloat16)
    pre = _encode(x16, We16, b_enc.reshape(1, LATENT_DIM))
    tau = _thresholds(pre)
    latents, reconstructed = _decode(pre, tau, Wd16,
                                     b_dec.reshape(1, INPUT_DIM))
    return (reconstructed, latents, pre)
